# baseline (speedup 1.0000x reference)
import functools

import jax
import jax.numpy as jnp
from jax import lax
from jax.experimental import pallas as pl
from jax.experimental.pallas import tpu as pltpu
from jax.experimental.pallas import tpu_sc as plsc

N = 10000
NP = 10240
E = 320000
NC = 2
NS = 16
NW = NC * NS
CH = 100
NCH = 100
RPS = NP // NS
RB = 2048
G = NP // RB

_SC_PARAMS = pltpu.CompilerParams(use_tc_tiling_on_sc=False)


def _mesh():
  return plsc.VectorSubcoreMesh(
      core_axis_name="c", subcore_axis_name="s", num_cores=NC, num_subcores=NS)


def _sc_degrees(src3, dst3, ones_h, zeros16):
  @functools.partial(
      pl.kernel,
      out_type=jax.ShapeDtypeStruct((NC, 2, NP, 16), jnp.float32),
      mesh=_mesh(),
      compiler_params=_SC_PARAMS,
      scratch_types=[
          pltpu.VMEM((NCH, CH), jnp.int32),
          pltpu.VMEM((NCH, CH), jnp.int32),
          pltpu.VMEM((CH, 16), jnp.float32),
          pltpu.VMEM_SHARED((NP, 16), jnp.float32),
          pltpu.VMEM_SHARED((NP, 16), jnp.float32),
      ],
  )
  def k(src_hbm, dst_hbm, ones_hbm, zeros_hbm, out_hbm,
        idx_s, idx_d, ones_v, deg_s, deg_d):
    cid = lax.axis_index("c")
    sid = lax.axis_index("s")
    w = cid * NS + sid
    r0 = sid * RPS
    pltpu.sync_copy(zeros_hbm.at[pl.ds(r0, RPS)], deg_s.at[pl.ds(r0, RPS)])
    pltpu.sync_copy(zeros_hbm.at[pl.ds(r0, RPS)], deg_d.at[pl.ds(r0, RPS)])
    pltpu.sync_copy(ones_hbm, ones_v)
    pltpu.sync_copy(src_hbm.at[w], idx_s)
    pltpu.sync_copy(dst_hbm.at[w], idx_d)
    plsc.subcore_barrier()

    def step(j, c):
      pltpu.sync_copy(ones_v, deg_s.at[idx_s.at[j]], add=True)
      pltpu.sync_copy(ones_v, deg_d.at[idx_d.at[j]], add=True)
      return c

    lax.fori_loop(0, NCH, step, 0)
    plsc.subcore_barrier()
    pltpu.sync_copy(deg_s.at[pl.ds(r0, RPS)],
                    out_hbm.at[cid, 0, pl.ds(r0, RPS)])
    pltpu.sync_copy(deg_d.at[pl.ds(r0, RPS)],
                    out_hbm.at[cid, 1, pl.ds(r0, RPS)])

  return k(src3, dst3, ones_h, zeros16)


def _sc_aggregate(xs, src3, dst3, zeros_f, feat):
  @functools.partial(
      pl.kernel,
      out_type=jax.ShapeDtypeStruct((NC, NP, feat), jnp.float32),
      mesh=_mesh(),
      compiler_params=_SC_PARAMS,
      scratch_types=[
          pltpu.VMEM((NCH, CH), jnp.int32),
          pltpu.VMEM((NCH, CH), jnp.int32),
          pltpu.VMEM((CH, feat), jnp.float32),
          pltpu.VMEM((CH, feat), jnp.float32),
          pltpu.SemaphoreType.DMA,
          pltpu.SemaphoreType.DMA,
          pltpu.VMEM_SHARED((NP, feat), jnp.float32),
      ],
  )
  def k(xs_hbm, src_hbm, dst_hbm, zeros_hbm, out_hbm,
        idx_s, idx_d, rows0, rows1, sg0, sg1, acc):
    cid = lax.axis_index("c")
    sid = lax.axis_index("s")
    w = cid * NS + sid
    r0 = sid * RPS
    pltpu.sync_copy(zeros_hbm.at[pl.ds(r0, RPS)], acc.at[pl.ds(r0, RPS)])
    pltpu.sync_copy(src_hbm.at[w], idx_s)
    pltpu.sync_copy(dst_hbm.at[w], idx_d)
    plsc.subcore_barrier()
    pltpu.async_copy(xs_hbm.at[idx_s.at[0]], rows0, sg0)

    def step(i, c):
      j = 2 * i
      pltpu.make_async_copy(xs_hbm.at[idx_s.at[j]], rows0, sg0).wait()
      pltpu.async_copy(xs_hbm.at[idx_s.at[j + 1]], rows1, sg1)
      pltpu.sync_copy(rows0, acc.at[idx_d.at[j]], add=True)
      pltpu.make_async_copy(xs_hbm.at[idx_s.at[j + 1]], rows1, sg1).wait()
      j2 = jnp.where(j + 2 < NCH, j + 2, 0)
      pltpu.async_copy(xs_hbm.at[idx_s.at[j2]], rows0, sg0)
      pltpu.sync_copy(rows1, acc.at[idx_d.at[j + 1]], add=True)
      return c

    lax.fori_loop(0, NCH // 2, step, 0)
    pltpu.make_async_copy(xs_hbm.at[idx_s.at[0]], rows0, sg0).wait()
    plsc.subcore_barrier()
    pltpu.sync_copy(acc.at[pl.ds(r0, RPS)], out_hbm.at[cid, pl.ds(r0, RPS)])

  return k(xs, src3, dst3, zeros_f)


def _tc_prescale(parts, features):
  def body(p_ref, x_ref, xs_ref, nsb_ref, ndb_ref):
    outdeg = p_ref[0, 0, :, 0:1] + p_ref[1, 0, :, 0:1]
    indeg = p_ref[0, 1, :, 0:1] + p_ref[1, 1, :, 0:1]
    nsrc = lax.rsqrt(jnp.maximum(outdeg, 1.0))
    ndst = lax.rsqrt(jnp.maximum(indeg, 1.0))
    xs_ref[...] = x_ref[...] * nsrc
    nsb_ref[...] = jnp.broadcast_to(nsrc, (RB, 128))
    ndb_ref[...] = jnp.broadcast_to(ndst, (RB, 128))

  return pl.pallas_call(
      body,
      grid=(G,),
      in_specs=[
          pl.BlockSpec((NC, 2, RB, 16), lambda i: (0, 0, i, 0)),
          pl.BlockSpec((RB, 128), lambda i: (i, 0)),
      ],
      out_specs=[pl.BlockSpec((RB, 128), lambda i: (i, 0))] * 3,
      out_shape=[jax.ShapeDtypeStruct((NP, 128), jnp.float32)] * 3,
  )(parts, features)


def _tc_layer1(agg, ndb, nsb, W, b):
  def body(a_ref, ndb_ref, nsb_ref, w_ref, b_ref, o_ref):
    a = (a_ref[0] + a_ref[1]) * ndb_ref[...]
    h = jnp.dot(a, w_ref[...], preferred_element_type=jnp.float32)
    o_ref[...] = jnp.maximum(h + b_ref[...], 0.0) * nsb_ref[...]

  return pl.pallas_call(
      body,
      grid=(G,),
      in_specs=[
          pl.BlockSpec((NC, RB, 128), lambda i: (0, i, 0)),
          pl.BlockSpec((RB, 128), lambda i: (i, 0)),
          pl.BlockSpec((RB, 128), lambda i: (i, 0)),
          pl.BlockSpec((128, 128), lambda i: (0, 0)),
          pl.BlockSpec((1, 128), lambda i: (0, 0)),
      ],
      out_specs=pl.BlockSpec((RB, 128), lambda i: (i, 0)),
      out_shape=jax.ShapeDtypeStruct((NP, 128), jnp.float32),
  )(agg, ndb, nsb, W, b)


def _tc_layer2_proj(agg, ndb, nsb, W2, b2, W3p):
  def body(a_ref, ndb_ref, nsb_ref, w2_ref, b2_ref, w3_ref, o_ref):
    a = (a_ref[0] + a_ref[1]) * ndb_ref[...]
    h = jnp.dot(a, w2_ref[...], preferred_element_type=jnp.float32)
    h = jnp.maximum(h + b2_ref[...], 0.0)
    t = jnp.dot(h, w3_ref[...], preferred_element_type=jnp.float32)
    o_ref[...] = t * nsb_ref[...][:, :48]

  return pl.pallas_call(
      body,
      grid=(G,),
      in_specs=[
          pl.BlockSpec((NC, RB, 128), lambda i: (0, i, 0)),
          pl.BlockSpec((RB, 128), lambda i: (i, 0)),
          pl.BlockSpec((RB, 128), lambda i: (i, 0)),
          pl.BlockSpec((128, 128), lambda i: (0, 0)),
          pl.BlockSpec((1, 128), lambda i: (0, 0)),
          pl.BlockSpec((128, 48), lambda i: (0, 0)),
      ],
      out_specs=pl.BlockSpec((RB, 48), lambda i: (i, 0)),
      out_shape=jax.ShapeDtypeStruct((NP, 48), jnp.float32),
  )(agg, ndb, nsb, W2, b2, W3p)


def _tc_final(agg, ndb, b3p):
  def body(a_ref, ndb_ref, b_ref, o_ref):
    o_ref[...] = (a_ref[0] + a_ref[1]) * ndb_ref[...][:, :48] + b_ref[...]

  return pl.pallas_call(
      body,
      grid=(G,),
      in_specs=[
          pl.BlockSpec((NC, RB, 48), lambda i: (0, i, 0)),
          pl.BlockSpec((RB, 128), lambda i: (i, 0)),
          pl.BlockSpec((1, 48), lambda i: (0, 0)),
      ],
      out_specs=pl.BlockSpec((RB, 48), lambda i: (i, 0)),
      out_shape=jax.ShapeDtypeStruct((NP, 48), jnp.float32),
  )(agg, ndb, b3p)


def kernel(features, edge_index, W1, b1, W2, b2, W3, b3):
  src3 = edge_index[0].astype(jnp.int32).reshape(NW, NCH, CH)
  dst3 = edge_index[1].astype(jnp.int32).reshape(NW, NCH, CH)
  ones_h = jnp.ones((CH, 16), jnp.float32)
  z16 = jnp.zeros((NP, 16), jnp.float32)
  z128 = jnp.zeros((NP, 128), jnp.float32)
  z48 = jnp.zeros((NP, 48), jnp.float32)
  fpad = jnp.pad(features, ((0, NP - N), (0, 0)))

  parts = _sc_degrees(src3, dst3, ones_h, z16)
  xs, nsb, ndb = _tc_prescale(parts, fpad)

  agg1 = _sc_aggregate(xs, src3, dst3, z128, 128)
  h1s = _tc_layer1(agg1, ndb, nsb, W1, b1.reshape(1, 128))

  agg2 = _sc_aggregate(h1s, src3, dst3, z128, 128)
  W3p = jnp.pad(W3, ((0, 0), (0, 8)))
  b3p = jnp.pad(b3, (0, 8)).reshape(1, 48)
  t3s = _tc_layer2_proj(agg2, ndb, nsb, W2, b2.reshape(1, 128), W3p)

  agg3 = _sc_aggregate(t3s, src3, dst3, z48, 48)
  out48 = _tc_final(agg3, ndb, b3p)
  return out48[:N, :40]


# speedup vs baseline: 1.0687x; 1.0687x over previous
import functools

import jax
import jax.numpy as jnp
from jax import lax
from jax.experimental import pallas as pl
from jax.experimental.pallas import tpu as pltpu
from jax.experimental.pallas import tpu_sc as plsc

N = 10000
NP = 10240
E = 320000
NC = 2
NS = 16
NW = NC * NS
CH = 125
NCH = 80
RPS = NP // NS
RB = 2048
G = NP // RB

_SC_PARAMS = pltpu.CompilerParams(use_tc_tiling_on_sc=False)


def _mesh():
  return plsc.VectorSubcoreMesh(
      core_axis_name="c", subcore_axis_name="s", num_cores=NC, num_subcores=NS)


def _sc_degrees(src3, dst3, ones_h, zeros16):
  @functools.partial(
      pl.kernel,
      out_type=jax.ShapeDtypeStruct((NC, 2, NP, 16), jnp.float32),
      mesh=_mesh(),
      compiler_params=_SC_PARAMS,
      scratch_types=[
          pltpu.VMEM((NCH, CH), jnp.int32),
          pltpu.VMEM((NCH, CH), jnp.int32),
          pltpu.VMEM((CH, 16), jnp.float32),
          pltpu.VMEM_SHARED((NP, 16), jnp.float32),
          pltpu.VMEM_SHARED((NP, 16), jnp.float32),
      ],
  )
  def k(src_hbm, dst_hbm, ones_hbm, zeros_hbm, out_hbm,
        idx_s, idx_d, ones_v, deg_s, deg_d):
    cid = lax.axis_index("c")
    sid = lax.axis_index("s")
    w = cid * NS + sid
    r0 = sid * RPS
    pltpu.sync_copy(zeros_hbm.at[pl.ds(r0, RPS)], deg_s.at[pl.ds(r0, RPS)])
    pltpu.sync_copy(zeros_hbm.at[pl.ds(r0, RPS)], deg_d.at[pl.ds(r0, RPS)])
    pltpu.sync_copy(ones_hbm, ones_v)
    pltpu.sync_copy(src_hbm.at[w], idx_s)
    pltpu.sync_copy(dst_hbm.at[w], idx_d)
    plsc.subcore_barrier()

    def step(j, c):
      pltpu.sync_copy(ones_v, deg_s.at[idx_s.at[j]], add=True)
      pltpu.sync_copy(ones_v, deg_d.at[idx_d.at[j]], add=True)
      return c

    lax.fori_loop(0, NCH, step, 0)
    plsc.subcore_barrier()
    pltpu.sync_copy(deg_s.at[pl.ds(r0, RPS)],
                    out_hbm.at[cid, 0, pl.ds(r0, RPS)])
    pltpu.sync_copy(deg_d.at[pl.ds(r0, RPS)],
                    out_hbm.at[cid, 1, pl.ds(r0, RPS)])

  return k(src3, dst3, ones_h, zeros16)


def _sc_aggregate(xs, src3, dst3, zeros_f, feat):
  @functools.partial(
      pl.kernel,
      out_type=jax.ShapeDtypeStruct((NC, NP, feat), jnp.float32),
      mesh=_mesh(),
      compiler_params=_SC_PARAMS,
      scratch_types=[
          pltpu.VMEM((NCH, CH), jnp.int32),
          pltpu.VMEM((CH,), jnp.int32),
          pltpu.VMEM((CH,), jnp.int32),
          pltpu.VMEM((CH, feat), jnp.float32),
          pltpu.VMEM((CH, feat), jnp.float32),
          pltpu.SemaphoreType.DMA,
          pltpu.SemaphoreType.DMA,
          pltpu.SemaphoreType.DMA,
          pltpu.SemaphoreType.DMA,
          pltpu.SemaphoreType.DMA,
          pltpu.SemaphoreType.DMA,
          pltpu.VMEM_SHARED((NP, feat), jnp.float32),
      ],
  )
  def k(xs_hbm, src_hbm, dst_hbm, zeros_hbm, out_hbm,
        idx_s, id0, id1, rows0, rows1, sg0, sg1, ss0, ss1, si0, si1, acc):
    cid = lax.axis_index("c")
    sid = lax.axis_index("s")
    w = cid * NS + sid
    r0 = sid * RPS
    pltpu.sync_copy(zeros_hbm.at[pl.ds(r0, RPS)], acc.at[pl.ds(r0, RPS)])
    pltpu.sync_copy(src_hbm.at[w], idx_s)
    pltpu.sync_copy(dst_hbm.at[w, NCH - 1], id1)
    pltpu.sync_copy(zeros_hbm.at[pl.ds(0, CH)], rows1)
    plsc.subcore_barrier()
    pltpu.async_copy(xs_hbm.at[idx_s.at[0]], rows0, sg0)
    pltpu.async_copy(rows1, acc.at[id1], ss1, add=True)
    pltpu.async_copy(dst_hbm.at[w, 0], id0, si0)

    def step(i, c):
      j = 2 * i
      pltpu.make_async_copy(xs_hbm.at[idx_s.at[j]], rows0, sg0).wait()
      pltpu.make_async_copy(rows1, acc.at[id1], ss1).wait()
      pltpu.async_copy(xs_hbm.at[idx_s.at[j + 1]], rows1, sg1)
      pltpu.make_async_copy(dst_hbm.at[w, 0], id0, si0).wait()
      pltpu.async_copy(rows0, acc.at[id0], ss0, add=True)
      pltpu.async_copy(dst_hbm.at[w, j + 1], id1, si1)
      pltpu.make_async_copy(xs_hbm.at[idx_s.at[j + 1]], rows1, sg1).wait()
      pltpu.make_async_copy(rows0, acc.at[id0], ss0).wait()
      j2 = jnp.where(j + 2 < NCH, j + 2, 0)
      pltpu.async_copy(xs_hbm.at[idx_s.at[j2]], rows0, sg0)
      pltpu.make_async_copy(dst_hbm.at[w, 0], id1, si1).wait()
      pltpu.async_copy(rows1, acc.at[id1], ss1, add=True)
      pltpu.async_copy(dst_hbm.at[w, j2], id0, si0)
      return c

    lax.fori_loop(0, NCH // 2, step, 0)
    pltpu.make_async_copy(xs_hbm.at[idx_s.at[0]], rows0, sg0).wait()
    pltpu.make_async_copy(rows1, acc.at[id1], ss1).wait()
    pltpu.make_async_copy(dst_hbm.at[w, 0], id0, si0).wait()
    plsc.subcore_barrier()
    pltpu.sync_copy(acc.at[pl.ds(r0, RPS)], out_hbm.at[cid, pl.ds(r0, RPS)])

  return k(xs, src3, dst3, zeros_f)


def _tc_prescale(parts, features):
  def body(p_ref, x_ref, xs_ref, nsb_ref, ndb_ref):
    outdeg = p_ref[0, 0, :, 0:1] + p_ref[1, 0, :, 0:1]
    indeg = p_ref[0, 1, :, 0:1] + p_ref[1, 1, :, 0:1]
    nsrc = lax.rsqrt(jnp.maximum(outdeg, 1.0))
    ndst = lax.rsqrt(jnp.maximum(indeg, 1.0))
    xs_ref[...] = x_ref[...] * nsrc
    nsb_ref[...] = jnp.broadcast_to(nsrc, (RB, 128))
    ndb_ref[...] = jnp.broadcast_to(ndst, (RB, 128))

  return pl.pallas_call(
      body,
      grid=(G,),
      in_specs=[
          pl.BlockSpec((NC, 2, RB, 16), lambda i: (0, 0, i, 0)),
          pl.BlockSpec((RB, 128), lambda i: (i, 0)),
      ],
      out_specs=[pl.BlockSpec((RB, 128), lambda i: (i, 0))] * 3,
      out_shape=[jax.ShapeDtypeStruct((NP, 128), jnp.float32)] * 3,
  )(parts, features)


def _tc_layer1(agg, ndb, nsb, W, b):
  def body(a_ref, ndb_ref, nsb_ref, w_ref, b_ref, o_ref):
    a = (a_ref[0] + a_ref[1]) * ndb_ref[...]
    h = jnp.dot(a, w_ref[...], preferred_element_type=jnp.float32)
    o_ref[...] = jnp.maximum(h + b_ref[...], 0.0) * nsb_ref[...]

  return pl.pallas_call(
      body,
      grid=(G,),
      in_specs=[
          pl.BlockSpec((NC, RB, 128), lambda i: (0, i, 0)),
          pl.BlockSpec((RB, 128), lambda i: (i, 0)),
          pl.BlockSpec((RB, 128), lambda i: (i, 0)),
          pl.BlockSpec((128, 128), lambda i: (0, 0)),
          pl.BlockSpec((1, 128), lambda i: (0, 0)),
      ],
      out_specs=pl.BlockSpec((RB, 128), lambda i: (i, 0)),
      out_shape=jax.ShapeDtypeStruct((NP, 128), jnp.float32),
  )(agg, ndb, nsb, W, b)


def _tc_layer2_proj(agg, ndb, nsb, W2, b2, W3p):
  def body(a_ref, ndb_ref, nsb_ref, w2_ref, b2_ref, w3_ref, o_ref):
    a = (a_ref[0] + a_ref[1]) * ndb_ref[...]
    h = jnp.dot(a, w2_ref[...], preferred_element_type=jnp.float32)
    h = jnp.maximum(h + b2_ref[...], 0.0)
    t = jnp.dot(h, w3_ref[...], preferred_element_type=jnp.float32)
    o_ref[...] = t * nsb_ref[...][:, :48]

  return pl.pallas_call(
      body,
      grid=(G,),
      in_specs=[
          pl.BlockSpec((NC, RB, 128), lambda i: (0, i, 0)),
          pl.BlockSpec((RB, 128), lambda i: (i, 0)),
          pl.BlockSpec((RB, 128), lambda i: (i, 0)),
          pl.BlockSpec((128, 128), lambda i: (0, 0)),
          pl.BlockSpec((1, 128), lambda i: (0, 0)),
          pl.BlockSpec((128, 48), lambda i: (0, 0)),
      ],
      out_specs=pl.BlockSpec((RB, 48), lambda i: (i, 0)),
      out_shape=jax.ShapeDtypeStruct((NP, 48), jnp.float32),
  )(agg, ndb, nsb, W2, b2, W3p)


def _tc_final(agg, ndb, b3p):
  def body(a_ref, ndb_ref, b_ref, o_ref):
    o_ref[...] = (a_ref[0] + a_ref[1]) * ndb_ref[...][:, :48] + b_ref[...]

  return pl.pallas_call(
      body,
      grid=(G,),
      in_specs=[
          pl.BlockSpec((NC, RB, 48), lambda i: (0, i, 0)),
          pl.BlockSpec((RB, 128), lambda i: (i, 0)),
          pl.BlockSpec((1, 48), lambda i: (0, 0)),
      ],
      out_specs=pl.BlockSpec((RB, 48), lambda i: (i, 0)),
      out_shape=jax.ShapeDtypeStruct((NP, 48), jnp.float32),
  )(agg, ndb, b3p)


def kernel(features, edge_index, W1, b1, W2, b2, W3, b3):
  src3 = edge_index[0].astype(jnp.int32).reshape(NW, NCH, CH)
  dst3 = edge_index[1].astype(jnp.int32).reshape(NW, NCH, CH)
  ones_h = jnp.ones((CH, 16), jnp.float32)
  z16 = jnp.zeros((NP, 16), jnp.float32)
  z128 = jnp.zeros((NP, 128), jnp.float32)
  z48 = jnp.zeros((NP, 48), jnp.float32)
  fpad = jnp.pad(features, ((0, NP - N), (0, 0)))

  parts = _sc_degrees(src3, dst3, ones_h, z16)
  xs, nsb, ndb = _tc_prescale(parts, fpad)

  agg1 = _sc_aggregate(xs, src3, dst3, z128, 128)
  h1s = _tc_layer1(agg1, ndb, nsb, W1, b1.reshape(1, 128))

  agg2 = _sc_aggregate(h1s, src3, dst3, z128, 128)
  W3p = jnp.pad(W3, ((0, 0), (0, 8)))
  b3p = jnp.pad(b3, (0, 8)).reshape(1, 48)
  t3s = _tc_layer2_proj(agg2, ndb, nsb, W2, b2.reshape(1, 128), W3p)

  agg3 = _sc_aggregate(t3s, src3, dst3, z48, 48)
  out48 = _tc_final(agg3, ndb, b3p)
  return out48[:N, :40]


# speedup vs baseline: 1.1082x; 1.0370x over previous
import functools

import jax
import jax.numpy as jnp
from jax import lax
from jax.experimental import pallas as pl
from jax.experimental.pallas import tpu as pltpu
from jax.experimental.pallas import tpu_sc as plsc

N = 10000
NP = 10240
E = 320000
NC = 2
NS = 16
NW = NC * NS
CH = 125
NCH = 80
RPS = NP // NS
RB = 2048
G = NP // RB

_SC_PARAMS = pltpu.CompilerParams(use_tc_tiling_on_sc=False)


def _mesh():
  return plsc.VectorSubcoreMesh(
      core_axis_name="c", subcore_axis_name="s", num_cores=NC, num_subcores=NS)


def _sc_degrees(src3, dst3, ones_h, zeros16):
  @functools.partial(
      pl.kernel,
      out_type=jax.ShapeDtypeStruct((NC, 2, NP, 16), jnp.float32),
      mesh=_mesh(),
      compiler_params=_SC_PARAMS,
      scratch_types=[
          pltpu.VMEM((NCH, CH), jnp.int32),
          pltpu.VMEM((NCH, CH), jnp.int32),
          pltpu.VMEM((CH, 16), jnp.float32),
          pltpu.VMEM_SHARED((NP, 16), jnp.float32),
          pltpu.VMEM_SHARED((NP, 16), jnp.float32),
      ],
  )
  def k(src_hbm, dst_hbm, ones_hbm, zeros_hbm, out_hbm,
        idx_s, idx_d, ones_v, deg_s, deg_d):
    cid = lax.axis_index("c")
    sid = lax.axis_index("s")
    w = cid * NS + sid
    r0 = sid * RPS
    pltpu.sync_copy(zeros_hbm.at[pl.ds(r0, RPS)], deg_s.at[pl.ds(r0, RPS)])
    pltpu.sync_copy(zeros_hbm.at[pl.ds(r0, RPS)], deg_d.at[pl.ds(r0, RPS)])
    pltpu.sync_copy(ones_hbm, ones_v)
    pltpu.sync_copy(src_hbm.at[w], idx_s)
    pltpu.sync_copy(dst_hbm.at[w], idx_d)
    plsc.subcore_barrier()

    def step(j, c):
      pltpu.sync_copy(ones_v, deg_s.at[idx_s.at[j]], add=True)
      pltpu.sync_copy(ones_v, deg_d.at[idx_d.at[j]], add=True)
      return c

    lax.fori_loop(0, NCH, step, 0)
    plsc.subcore_barrier()
    pltpu.sync_copy(deg_s.at[pl.ds(r0, RPS)],
                    out_hbm.at[cid, 0, pl.ds(r0, RPS)])
    pltpu.sync_copy(deg_d.at[pl.ds(r0, RPS)],
                    out_hbm.at[cid, 1, pl.ds(r0, RPS)])

  return k(src3, dst3, ones_h, zeros16)


def _sc_aggregate(xs, src3, dst3, zeros_f, feat):
  @functools.partial(
      pl.kernel,
      out_type=jax.ShapeDtypeStruct((NC, NP, feat), jnp.float32),
      mesh=_mesh(),
      compiler_params=_SC_PARAMS,
      scratch_types=[
          pltpu.VMEM((NCH, CH), jnp.int32),
          pltpu.VMEM((CH,), jnp.int32),
          pltpu.VMEM((CH,), jnp.int32),
          pltpu.VMEM((CH, feat), jnp.float32),
          pltpu.VMEM((CH, feat), jnp.float32),
          pltpu.SemaphoreType.DMA,
          pltpu.SemaphoreType.DMA,
          pltpu.SemaphoreType.DMA,
          pltpu.SemaphoreType.DMA,
          pltpu.SemaphoreType.DMA,
          pltpu.SemaphoreType.DMA,
          pltpu.VMEM_SHARED((NP, feat), jnp.float32),
      ],
  )
  def k(xs_hbm, src_hbm, dst_hbm, zeros_hbm, out_hbm,
        idx_s, id0, id1, rows0, rows1, sg0, sg1, ss0, ss1, si0, si1, acc):
    cid = lax.axis_index("c")
    sid = lax.axis_index("s")
    w = cid * NS + sid
    r0 = sid * RPS
    pltpu.sync_copy(zeros_hbm.at[pl.ds(r0, RPS)], acc.at[pl.ds(r0, RPS)])
    pltpu.sync_copy(src_hbm.at[w], idx_s)
    pltpu.sync_copy(dst_hbm.at[w, NCH - 1], id1)
    pltpu.sync_copy(zeros_hbm.at[pl.ds(0, CH)], rows1)
    plsc.subcore_barrier()
    pltpu.async_copy(xs_hbm.at[idx_s.at[0]], rows0, sg0)
    pltpu.async_copy(rows1, acc.at[id1], ss1, add=True)
    pltpu.async_copy(dst_hbm.at[w, 0], id0, si0)

    def step(i, c):
      j = 2 * i
      pltpu.make_async_copy(xs_hbm.at[idx_s.at[j]], rows0, sg0).wait()
      pltpu.make_async_copy(rows1, acc.at[id1], ss1).wait()
      pltpu.async_copy(xs_hbm.at[idx_s.at[j + 1]], rows1, sg1)
      pltpu.make_async_copy(dst_hbm.at[w, 0], id0, si0).wait()
      pltpu.async_copy(rows0, acc.at[id0], ss0, add=True)
      pltpu.async_copy(dst_hbm.at[w, j + 1], id1, si1)
      pltpu.make_async_copy(xs_hbm.at[idx_s.at[j + 1]], rows1, sg1).wait()
      pltpu.make_async_copy(rows0, acc.at[id0], ss0).wait()
      j2 = jnp.where(j + 2 < NCH, j + 2, 0)
      pltpu.async_copy(xs_hbm.at[idx_s.at[j2]], rows0, sg0)
      pltpu.make_async_copy(dst_hbm.at[w, 0], id1, si1).wait()
      pltpu.async_copy(rows1, acc.at[id1], ss1, add=True)
      pltpu.async_copy(dst_hbm.at[w, j2], id0, si0)
      return c

    lax.fori_loop(0, NCH // 2, step, 0)
    pltpu.make_async_copy(xs_hbm.at[idx_s.at[0]], rows0, sg0).wait()
    pltpu.make_async_copy(rows1, acc.at[id1], ss1).wait()
    pltpu.make_async_copy(dst_hbm.at[w, 0], id0, si0).wait()
    plsc.subcore_barrier()
    pltpu.sync_copy(acc.at[pl.ds(r0, RPS)], out_hbm.at[cid, pl.ds(r0, RPS)])

  return k(xs, src3, dst3, zeros_f)


def _tc_prescale(praw, features):
  RB8 = RB // 8

  def body(p_ref, x_ref, xs_ref, nsb_ref, ndb_ref):
    ii = lax.broadcasted_iota(jnp.int32, (128, 8), 0)
    kk = lax.broadcasted_iota(jnp.int32, (128, 8), 1)
    sel = (ii == kk * 16).astype(jnp.float32)
    ku = lax.broadcasted_iota(jnp.int32, (8, 1024), 0)
    mu = lax.broadcasted_iota(jnp.int32, (8, 1024), 1)
    spread = (mu // 128 == ku).astype(jnp.float32)

    def norm_table(t):
      y = (jnp.dot(p_ref[0, t], sel, preferred_element_type=jnp.float32)
           + jnp.dot(p_ref[1, t], sel, preferred_element_type=jnp.float32))
      v = lax.rsqrt(jnp.maximum(y, 1.0))
      b = jnp.dot(v, spread, preferred_element_type=jnp.float32)
      return b.reshape(RB, 128)

    nsb = norm_table(0)
    ndb = norm_table(1)
    xs_ref[...] = x_ref[...] * nsb
    nsb_ref[...] = nsb
    ndb_ref[...] = ndb

  return pl.pallas_call(
      body,
      grid=(G,),
      in_specs=[
          pl.BlockSpec((NC, 2, RB8, 128), lambda i: (0, 0, i, 0)),
          pl.BlockSpec((RB, 128), lambda i: (i, 0)),
      ],
      out_specs=[pl.BlockSpec((RB, 128), lambda i: (i, 0))] * 3,
      out_shape=[jax.ShapeDtypeStruct((NP, 128), jnp.float32)] * 3,
  )(praw, features)


def _tc_layer1(agg, ndb, nsb, W, b):
  def body(a_ref, ndb_ref, nsb_ref, w_ref, b_ref, o_ref):
    a = (a_ref[0] + a_ref[1]) * ndb_ref[...]
    h = jnp.dot(a, w_ref[...], preferred_element_type=jnp.float32)
    o_ref[...] = jnp.maximum(h + b_ref[...], 0.0) * nsb_ref[...]

  return pl.pallas_call(
      body,
      grid=(G,),
      in_specs=[
          pl.BlockSpec((NC, RB, 128), lambda i: (0, i, 0)),
          pl.BlockSpec((RB, 128), lambda i: (i, 0)),
          pl.BlockSpec((RB, 128), lambda i: (i, 0)),
          pl.BlockSpec((128, 128), lambda i: (0, 0)),
          pl.BlockSpec((1, 128), lambda i: (0, 0)),
      ],
      out_specs=pl.BlockSpec((RB, 128), lambda i: (i, 0)),
      out_shape=jax.ShapeDtypeStruct((NP, 128), jnp.float32),
  )(agg, ndb, nsb, W, b)


def _tc_layer2_proj(agg, ndb, nsb, W2, b2, W3p):
  def body(a_ref, ndb_ref, nsb_ref, w2_ref, b2_ref, w3_ref, o_ref):
    a = (a_ref[0] + a_ref[1]) * ndb_ref[...]
    h = jnp.dot(a, w2_ref[...], preferred_element_type=jnp.float32)
    h = jnp.maximum(h + b2_ref[...], 0.0)
    t = jnp.dot(h, w3_ref[...], preferred_element_type=jnp.float32)
    o_ref[...] = t * nsb_ref[...][:, :48]

  return pl.pallas_call(
      body,
      grid=(G,),
      in_specs=[
          pl.BlockSpec((NC, RB, 128), lambda i: (0, i, 0)),
          pl.BlockSpec((RB, 128), lambda i: (i, 0)),
          pl.BlockSpec((RB, 128), lambda i: (i, 0)),
          pl.BlockSpec((128, 128), lambda i: (0, 0)),
          pl.BlockSpec((1, 128), lambda i: (0, 0)),
          pl.BlockSpec((128, 48), lambda i: (0, 0)),
      ],
      out_specs=pl.BlockSpec((RB, 48), lambda i: (i, 0)),
      out_shape=jax.ShapeDtypeStruct((NP, 48), jnp.float32),
  )(agg, ndb, nsb, W2, b2, W3p)


def _tc_final(agg, ndb, b3p):
  def body(a_ref, ndb_ref, b_ref, o_ref):
    o_ref[...] = (a_ref[0] + a_ref[1]) * ndb_ref[...][:, :48] + b_ref[...]

  return pl.pallas_call(
      body,
      grid=(G,),
      in_specs=[
          pl.BlockSpec((NC, RB, 48), lambda i: (0, i, 0)),
          pl.BlockSpec((RB, 128), lambda i: (i, 0)),
          pl.BlockSpec((1, 48), lambda i: (0, 0)),
      ],
      out_specs=pl.BlockSpec((RB, 48), lambda i: (i, 0)),
      out_shape=jax.ShapeDtypeStruct((NP, 48), jnp.float32),
  )(agg, ndb, b3p)


def kernel(features, edge_index, W1, b1, W2, b2, W3, b3):
  src3 = edge_index[0].astype(jnp.int32).reshape(NW, NCH, CH)
  dst3 = edge_index[1].astype(jnp.int32).reshape(NW, NCH, CH)
  ones_h = jnp.ones((CH, 16), jnp.float32)
  z16 = jnp.zeros((NP, 16), jnp.float32)
  z128 = jnp.zeros((NP, 128), jnp.float32)
  z48 = jnp.zeros((NP, 48), jnp.float32)
  fpad = jnp.pad(features, ((0, NP - N), (0, 0)))

  parts = _sc_degrees(src3, dst3, ones_h, z16)
  praw = parts.reshape(NC, 2, NP // 8, 128)
  xs, nsb, ndb = _tc_prescale(praw, fpad)

  agg1 = _sc_aggregate(xs, src3, dst3, z128, 128)
  h1s = _tc_layer1(agg1, ndb, nsb, W1, b1.reshape(1, 128))

  agg2 = _sc_aggregate(h1s, src3, dst3, z128, 128)
  W3p = jnp.pad(W3, ((0, 0), (0, 8)))
  b3p = jnp.pad(b3, (0, 8)).reshape(1, 48)
  t3s = _tc_layer2_proj(agg2, ndb, nsb, W2, b2.reshape(1, 128), W3p)

  agg3 = _sc_aggregate(t3s, src3, dst3, z48, 48)
  out48 = _tc_final(agg3, ndb, b3p)
  return out48[:N, :40]
